# Initial kernel scaffold; baseline (speedup 1.0000x reference)
import jax
import jax.numpy as jnp
from jax.experimental import pallas as pl


def kernel(x, edge_index, W_l, b_l, W_r, gamma, beta):
    raise NotImplementedError("write your pallas kernel here")


# speedup vs baseline: 7.3693x; 7.3693x over previous
import functools

import jax
import jax.numpy as jnp
from jax import lax
from jax.experimental import pallas as pl
from jax.experimental.pallas import tpu as pltpu
from jax.experimental.pallas import tpu_sc as plsc

NUM_CORES = 2
NUM_SUBCORES = 16
NW = NUM_CORES * NUM_SUBCORES
K = 128
G = 8


def _row_chunks(rows_per_tile):
    chunks = []
    off = 0
    while off < rows_per_tile:
        sz = min(K, rows_per_tile - off)
        chunks.append((off, sz))
        off += sz
    return chunks


def _sc_sum(x, src4, dst4, n_pad, groups):
    n, d = x.shape
    rows_per_tile = n_pad // NUM_SUBCORES
    row_chunks = _row_chunks(rows_per_tile)
    z_rows = jnp.zeros((K, d), jnp.float32)
    mesh = plsc.VectorSubcoreMesh(core_axis_name="c", subcore_axis_name="s")

    @functools.partial(
        pl.kernel,
        out_type=jax.ShapeDtypeStruct((NUM_CORES, n_pad, d), jnp.float32),
        mesh=mesh,
        scratch_types=[
            pltpu.VMEM((G, K), jnp.int32),
            pltpu.VMEM((G, K), jnp.int32),
            pltpu.VMEM((K, d), jnp.float32),
            pltpu.VMEM_SHARED((n_pad, d), jnp.float32),
            pltpu.SemaphoreType.DMA,
        ],
    )
    def sc_kernel(x_hbm, src_hbm, dst_hbm, zr_hbm, acc_out,
                  src_v, dst_v, rows_v, acc_sh, sem):
        c = lax.axis_index("c")
        s = lax.axis_index("s")
        wid = s * NUM_CORES + c
        row0 = s * rows_per_tile

        pltpu.sync_copy(zr_hbm, rows_v)
        for off, sz in row_chunks:
            pltpu.sync_copy(rows_v.at[pl.ds(0, sz)],
                            acc_sh.at[pl.ds(row0 + off, sz)])
        plsc.subcore_barrier()

        @pl.loop(0, groups)
        def _(g):
            pltpu.sync_copy(src_hbm.at[wid, g], src_v)
            pltpu.sync_copy(dst_hbm.at[wid, g], dst_v)
            for jj in range(G):
                pltpu.async_copy(x_hbm.at[src_v.at[jj]], rows_v, sem).wait()
                pltpu.sync_copy(rows_v, acc_sh.at[dst_v.at[jj]], add=True)

        plsc.subcore_barrier()

        for off, sz in row_chunks:
            pltpu.sync_copy(acc_sh.at[pl.ds(row0 + off, sz)],
                            rows_v.at[pl.ds(0, sz)])
            pltpu.sync_copy(rows_v.at[pl.ds(0, sz)],
                            acc_out.at[c, pl.ds(row0 + off, sz)])

    return sc_kernel(x, src4, dst4, z_rows)


def _sc_count(dst4, n_pad, groups):
    rows_per_tile = n_pad // NUM_SUBCORES
    row_chunks = _row_chunks(rows_per_tile)
    z_rows = jnp.zeros((K, K), jnp.float32)
    o_rows = jnp.ones((K, K), jnp.float32)
    mesh = plsc.VectorSubcoreMesh(core_axis_name="c", subcore_axis_name="s")

    @functools.partial(
        pl.kernel,
        out_type=jax.ShapeDtypeStruct((NUM_CORES, n_pad, K), jnp.float32),
        mesh=mesh,
        scratch_types=[
            pltpu.VMEM((G, K), jnp.int32),
            pltpu.VMEM((K, K), jnp.float32),
            pltpu.VMEM_SHARED((n_pad, K), jnp.float32),
        ],
    )
    def sc_kernel(dst_hbm, zr_hbm, on_hbm, cnt_out,
                  dst_v, rows_v, acc_sh):
        c = lax.axis_index("c")
        s = lax.axis_index("s")
        wid = s * NUM_CORES + c
        row0 = s * rows_per_tile

        pltpu.sync_copy(zr_hbm, rows_v)
        for off, sz in row_chunks:
            pltpu.sync_copy(rows_v.at[pl.ds(0, sz)],
                            acc_sh.at[pl.ds(row0 + off, sz)])
        pltpu.sync_copy(on_hbm, rows_v)
        plsc.subcore_barrier()

        @pl.loop(0, groups)
        def _(g):
            pltpu.sync_copy(dst_hbm.at[wid, g], dst_v)
            for jj in range(G):
                pltpu.sync_copy(rows_v, acc_sh.at[dst_v.at[jj]], add=True)

        plsc.subcore_barrier()

        for off, sz in row_chunks:
            pltpu.sync_copy(acc_sh.at[pl.ds(row0 + off, sz)],
                            rows_v.at[pl.ds(0, sz)])
            pltpu.sync_copy(rows_v.at[pl.ds(0, sz)],
                            cnt_out.at[c, pl.ds(row0 + off, sz)])

    return sc_kernel(dst4, z_rows, o_rows)


def _tc_finish_body(acc_ref, cnt_ref, x_ref, wlT_ref, wrT_ref, b_ref,
                    g_ref, be_ref, o_ref):
    summed = acc_ref[0] + acc_ref[1]
    cnt = cnt_ref[0][:, 0:1] + cnt_ref[1][:, 0:1]
    mean = summed / jnp.clip(cnt, 1.0, None)
    out = (jnp.dot(mean, wlT_ref[...], preferred_element_type=jnp.float32)
           + jnp.dot(x_ref[...], wrT_ref[...], preferred_element_type=jnp.float32)
           + b_ref[...])
    mu = jnp.mean(out, axis=-1, keepdims=True)
    var = jnp.mean((out - mu) ** 2, axis=-1, keepdims=True)
    out = (out - mu) / jnp.sqrt(var + 1e-5) * g_ref[...] + be_ref[...]
    o_ref[...] = jnp.maximum(out, 0.0)


def _tc_finish(acc, cnt, x, W_l, b_l, W_r, gamma, beta):
    n, d = x.shape
    blk = 2000
    grid = n // blk
    return pl.pallas_call(
        _tc_finish_body,
        grid=(grid,),
        in_specs=[
            pl.BlockSpec((NUM_CORES, blk, d), lambda i: (0, i, 0)),
            pl.BlockSpec((NUM_CORES, blk, K), lambda i: (0, i, 0)),
            pl.BlockSpec((blk, d), lambda i: (i, 0)),
            pl.BlockSpec((d, d), lambda i: (0, 0)),
            pl.BlockSpec((d, d), lambda i: (0, 0)),
            pl.BlockSpec((1, d), lambda i: (0, 0)),
            pl.BlockSpec((1, d), lambda i: (0, 0)),
            pl.BlockSpec((1, d), lambda i: (0, 0)),
        ],
        out_specs=pl.BlockSpec((blk, d), lambda i: (i, 0)),
        out_shape=jax.ShapeDtypeStruct((n, d), jnp.float32),
    )(acc, cnt, x, W_l.T, W_r.T, b_l.reshape(1, d),
      gamma.reshape(1, d), beta.reshape(1, d))


def kernel(x, edge_index, W_l, b_l, W_r, gamma, beta):
    n, d = x.shape
    e = edge_index.shape[1]

    n_pad = -(-(n + 1) // K) * K

    epw = -(-e // (NW * G * K)) * (G * K)
    groups = epw // (G * K)
    e_pad = epw * NW
    pad = e_pad - e
    src = edge_index[0].astype(jnp.int32)
    dst = edge_index[1].astype(jnp.int32)
    if pad:
        spread = n + jnp.arange(pad, dtype=jnp.int32) % (n_pad - n)
        src = jnp.concatenate([src, jnp.arange(pad, dtype=jnp.int32) % n])
        dst = jnp.concatenate([dst, spread])
    src4 = src.reshape(NW, groups, G, K)
    dst4 = dst.reshape(NW, groups, G, K)

    acc = _sc_sum(x, src4, dst4, n_pad, groups)
    cnt = _sc_count(dst4, n_pad, groups)
    return _tc_finish(acc, cnt, x, W_l, b_l, W_r, gamma, beta)


# speedup vs baseline: 8.9962x; 1.2208x over previous
import functools

import jax
import jax.numpy as jnp
from jax import lax
from jax.experimental import pallas as pl
from jax.experimental.pallas import tpu as pltpu
from jax.experimental.pallas import tpu_sc as plsc

NUM_CORES = 2
NUM_SUBCORES = 16
NW = NUM_CORES * NUM_SUBCORES
K = 128
G = 8


def _row_chunks(rows_per_tile):
    chunks = []
    off = 0
    while off < rows_per_tile:
        sz = min(K, rows_per_tile - off)
        chunks.append((off, sz))
        off += sz
    return chunks


def _sc_sum(x, src4, dst4, n_pad, groups):
    n, d = x.shape
    rows_per_tile = n_pad // NUM_SUBCORES
    row_chunks = _row_chunks(rows_per_tile)
    z_rows = jnp.zeros((K, d), jnp.float32)
    mesh = plsc.VectorSubcoreMesh(core_axis_name="c", subcore_axis_name="s")

    @functools.partial(
        pl.kernel,
        out_type=jax.ShapeDtypeStruct((NUM_CORES, n_pad, d), jnp.float32),
        mesh=mesh,
        scratch_types=[
            pltpu.VMEM((G, K), jnp.int32),
            pltpu.VMEM((G, K), jnp.int32),
            pltpu.VMEM((K, d), jnp.float32),
            pltpu.VMEM((K, d), jnp.float32),
            pltpu.VMEM_SHARED((n_pad, d), jnp.float32),
            pltpu.SemaphoreType.DMA,
            pltpu.SemaphoreType.DMA,
        ],
    )
    def sc_kernel(x_hbm, src_hbm, dst_hbm, zr_hbm, acc_out,
                  src_v, dst_v, rows0_v, rows1_v, acc_sh, sem0, sem1):
        c = lax.axis_index("c")
        s = lax.axis_index("s")
        wid = s * NUM_CORES + c
        row0 = s * rows_per_tile
        bufs = (rows0_v, rows1_v)
        sems = (sem0, sem1)

        pltpu.sync_copy(zr_hbm, rows0_v)
        for off, sz in row_chunks:
            pltpu.sync_copy(rows0_v.at[pl.ds(0, sz)],
                            acc_sh.at[pl.ds(row0 + off, sz)])
        plsc.subcore_barrier()

        @pl.loop(0, groups)
        def _(g):
            pltpu.sync_copy(src_hbm.at[wid, g], src_v)
            pltpu.sync_copy(dst_hbm.at[wid, g], dst_v)
            pltpu.async_copy(x_hbm.at[src_v.at[0]], bufs[0], sems[0])
            for jj in range(G):
                if jj + 1 < G:
                    pltpu.async_copy(x_hbm.at[src_v.at[jj + 1]],
                                     bufs[(jj + 1) % 2], sems[(jj + 1) % 2])
                pltpu.make_async_copy(
                    x_hbm.at[src_v.at[jj]], bufs[jj % 2], sems[jj % 2]).wait()
                pltpu.sync_copy(bufs[jj % 2],
                                acc_sh.at[dst_v.at[jj]], add=True)

        plsc.subcore_barrier()

        for off, sz in row_chunks:
            pltpu.sync_copy(acc_sh.at[pl.ds(row0 + off, sz)],
                            rows0_v.at[pl.ds(0, sz)])
            pltpu.sync_copy(rows0_v.at[pl.ds(0, sz)],
                            acc_out.at[c, pl.ds(row0 + off, sz)])

    return sc_kernel(x, src4, dst4, z_rows)


def _sc_count(dst4, n_pad, groups):
    rows_per_tile = n_pad // NUM_SUBCORES
    row_chunks = _row_chunks(rows_per_tile)
    z_rows = jnp.zeros((K, K), jnp.float32)
    o_rows = jnp.ones((K, K), jnp.float32)
    mesh = plsc.VectorSubcoreMesh(core_axis_name="c", subcore_axis_name="s")

    @functools.partial(
        pl.kernel,
        out_type=jax.ShapeDtypeStruct((NUM_CORES, n_pad, K), jnp.float32),
        mesh=mesh,
        scratch_types=[
            pltpu.VMEM((G, K), jnp.int32),
            pltpu.VMEM((K, K), jnp.float32),
            pltpu.VMEM_SHARED((n_pad, K), jnp.float32),
        ],
    )
    def sc_kernel(dst_hbm, zr_hbm, on_hbm, cnt_out,
                  dst_v, rows_v, acc_sh):
        c = lax.axis_index("c")
        s = lax.axis_index("s")
        wid = s * NUM_CORES + c
        row0 = s * rows_per_tile

        pltpu.sync_copy(zr_hbm, rows_v)
        for off, sz in row_chunks:
            pltpu.sync_copy(rows_v.at[pl.ds(0, sz)],
                            acc_sh.at[pl.ds(row0 + off, sz)])
        pltpu.sync_copy(on_hbm, rows_v)
        plsc.subcore_barrier()

        @pl.loop(0, groups)
        def _(g):
            pltpu.sync_copy(dst_hbm.at[wid, g], dst_v)
            for jj in range(G):
                pltpu.sync_copy(rows_v, acc_sh.at[dst_v.at[jj]], add=True)

        plsc.subcore_barrier()

        for off, sz in row_chunks:
            pltpu.sync_copy(acc_sh.at[pl.ds(row0 + off, sz)],
                            rows_v.at[pl.ds(0, sz)])
            pltpu.sync_copy(rows_v.at[pl.ds(0, sz)],
                            cnt_out.at[c, pl.ds(row0 + off, sz)])

    return sc_kernel(dst4, z_rows, o_rows)


def _tc_finish_body(acc_ref, cnt_ref, x_ref, wlT_ref, wrT_ref, b_ref,
                    g_ref, be_ref, o_ref):
    summed = acc_ref[0] + acc_ref[1]
    cnt = cnt_ref[0][:, 0:1] + cnt_ref[1][:, 0:1]
    mean = summed / jnp.clip(cnt, 1.0, None)
    out = (jnp.dot(mean, wlT_ref[...], preferred_element_type=jnp.float32)
           + jnp.dot(x_ref[...], wrT_ref[...], preferred_element_type=jnp.float32)
           + b_ref[...])
    mu = jnp.mean(out, axis=-1, keepdims=True)
    var = jnp.mean((out - mu) ** 2, axis=-1, keepdims=True)
    out = (out - mu) / jnp.sqrt(var + 1e-5) * g_ref[...] + be_ref[...]
    o_ref[...] = jnp.maximum(out, 0.0)


def _tc_finish(acc, cnt, x, W_l, b_l, W_r, gamma, beta):
    n, d = x.shape
    blk = 2000
    grid = n // blk
    return pl.pallas_call(
        _tc_finish_body,
        grid=(grid,),
        in_specs=[
            pl.BlockSpec((NUM_CORES, blk, d), lambda i: (0, i, 0)),
            pl.BlockSpec((NUM_CORES, blk, K), lambda i: (0, i, 0)),
            pl.BlockSpec((blk, d), lambda i: (i, 0)),
            pl.BlockSpec((d, d), lambda i: (0, 0)),
            pl.BlockSpec((d, d), lambda i: (0, 0)),
            pl.BlockSpec((1, d), lambda i: (0, 0)),
            pl.BlockSpec((1, d), lambda i: (0, 0)),
            pl.BlockSpec((1, d), lambda i: (0, 0)),
        ],
        out_specs=pl.BlockSpec((blk, d), lambda i: (i, 0)),
        out_shape=jax.ShapeDtypeStruct((n, d), jnp.float32),
    )(acc, cnt, x, W_l.T, W_r.T, b_l.reshape(1, d),
      gamma.reshape(1, d), beta.reshape(1, d))


def kernel(x, edge_index, W_l, b_l, W_r, gamma, beta):
    n, d = x.shape
    e = edge_index.shape[1]

    n_pad = -(-(n + 1) // K) * K

    epw = -(-e // (NW * G * K)) * (G * K)
    groups = epw // (G * K)
    e_pad = epw * NW
    pad = e_pad - e
    src = edge_index[0].astype(jnp.int32)
    dst = edge_index[1].astype(jnp.int32)
    if pad:
        spread = n + jnp.arange(pad, dtype=jnp.int32) % (n_pad - n)
        src = jnp.concatenate([src, jnp.arange(pad, dtype=jnp.int32) % n])
        dst = jnp.concatenate([dst, spread])
    src4 = src.reshape(NW, groups, G, K)
    dst4 = dst.reshape(NW, groups, G, K)

    acc = _sc_sum(x, src4, dst4, n_pad, groups)
    cnt = _sc_count(dst4, n_pad, groups)
    return _tc_finish(acc, cnt, x, W_l, b_l, W_r, gamma, beta)


# speedup vs baseline: 9.4660x; 1.0522x over previous
import functools

import jax
import jax.numpy as jnp
from jax import lax
from jax.experimental import pallas as pl
from jax.experimental.pallas import tpu as pltpu
from jax.experimental.pallas import tpu_sc as plsc

NUM_CORES = 2
NUM_SUBCORES = 16
NW = NUM_CORES * NUM_SUBCORES
K = 128
G = 16


def _row_chunks(rows_per_tile):
    chunks = []
    off = 0
    while off < rows_per_tile:
        sz = min(K, rows_per_tile - off)
        chunks.append((off, sz))
        off += sz
    return chunks


def _sc_sum(x, src4, dst4, n_pad, groups):
    n, d = x.shape
    rows_per_tile = n_pad // NUM_SUBCORES
    row_chunks = _row_chunks(rows_per_tile)
    z_rows = jnp.zeros((K, d), jnp.float32)
    mesh = plsc.VectorSubcoreMesh(core_axis_name="c", subcore_axis_name="s")

    @functools.partial(
        pl.kernel,
        out_type=jax.ShapeDtypeStruct((NUM_CORES, n_pad, d), jnp.float32),
        mesh=mesh,
        scratch_types=[
            pltpu.VMEM((G, K), jnp.int32),
            pltpu.VMEM((G, K), jnp.int32),
            pltpu.VMEM((K, d), jnp.float32),
            pltpu.VMEM((K, d), jnp.float32),
            pltpu.VMEM_SHARED((n_pad, d), jnp.float32),
            pltpu.SemaphoreType.DMA,
            pltpu.SemaphoreType.DMA,
        ],
    )
    def sc_kernel(x_hbm, src_hbm, dst_hbm, zr_hbm, acc_out,
                  src_v, dst_v, rows0_v, rows1_v, acc_sh, sem0, sem1):
        c = lax.axis_index("c")
        s = lax.axis_index("s")
        wid = s * NUM_CORES + c
        row0 = s * rows_per_tile
        bufs = (rows0_v, rows1_v)
        sems = (sem0, sem1)

        pltpu.sync_copy(zr_hbm, rows0_v)
        for off, sz in row_chunks:
            pltpu.sync_copy(rows0_v.at[pl.ds(0, sz)],
                            acc_sh.at[pl.ds(row0 + off, sz)])
        plsc.subcore_barrier()

        @pl.loop(0, groups)
        def _(g):
            pltpu.sync_copy(src_hbm.at[wid, g], src_v)
            pltpu.sync_copy(dst_hbm.at[wid, g], dst_v)
            pltpu.async_copy(x_hbm.at[src_v.at[0]], bufs[0], sems[0])
            for jj in range(G):
                if jj + 1 < G:
                    pltpu.async_copy(x_hbm.at[src_v.at[jj + 1]],
                                     bufs[(jj + 1) % 2], sems[(jj + 1) % 2])
                pltpu.make_async_copy(
                    x_hbm.at[src_v.at[jj]], bufs[jj % 2], sems[jj % 2]).wait()
                pltpu.sync_copy(bufs[jj % 2],
                                acc_sh.at[dst_v.at[jj]], add=True)

        plsc.subcore_barrier()

        for off, sz in row_chunks:
            pltpu.sync_copy(acc_sh.at[pl.ds(row0 + off, sz)],
                            rows0_v.at[pl.ds(0, sz)])
            pltpu.sync_copy(rows0_v.at[pl.ds(0, sz)],
                            acc_out.at[c, pl.ds(row0 + off, sz)])

    return sc_kernel(x, src4, dst4, z_rows)


CW = 128


def _sc_count(dst4, n_pad, groups):
    rows_per_tile = n_pad // NUM_SUBCORES
    row_chunks = _row_chunks(rows_per_tile)
    z_rows = jnp.zeros((K, CW), jnp.float32)
    o_rows = jnp.ones((K, CW), jnp.float32)
    mesh = plsc.VectorSubcoreMesh(core_axis_name="c", subcore_axis_name="s")

    @functools.partial(
        pl.kernel,
        out_type=jax.ShapeDtypeStruct((NUM_CORES, n_pad, CW), jnp.float32),
        mesh=mesh,
        scratch_types=[
            pltpu.VMEM((G, K), jnp.int32),
            pltpu.VMEM((K, CW), jnp.float32),
            pltpu.VMEM_SHARED((n_pad, CW), jnp.float32),
        ],
    )
    def sc_kernel(dst_hbm, zr_hbm, on_hbm, cnt_out,
                  dst_v, rows_v, acc_sh):
        c = lax.axis_index("c")
        s = lax.axis_index("s")
        wid = s * NUM_CORES + c
        row0 = s * rows_per_tile

        pltpu.sync_copy(zr_hbm, rows_v)
        for off, sz in row_chunks:
            pltpu.sync_copy(rows_v.at[pl.ds(0, sz)],
                            acc_sh.at[pl.ds(row0 + off, sz)])
        pltpu.sync_copy(on_hbm, rows_v)
        plsc.subcore_barrier()

        @pl.loop(0, groups)
        def _(g):
            pltpu.sync_copy(dst_hbm.at[wid, g], dst_v)
            for jj in range(G):
                pltpu.sync_copy(rows_v, acc_sh.at[dst_v.at[jj]], add=True)

        plsc.subcore_barrier()

        for off, sz in row_chunks:
            pltpu.sync_copy(acc_sh.at[pl.ds(row0 + off, sz)],
                            rows_v.at[pl.ds(0, sz)])
            pltpu.sync_copy(rows_v.at[pl.ds(0, sz)],
                            cnt_out.at[c, pl.ds(row0 + off, sz)])

    return sc_kernel(dst4, z_rows, o_rows)


def _tc_finish_body(acc_ref, cnt_ref, x_ref, wlT_ref, wrT_ref, b_ref,
                    g_ref, be_ref, o_ref):
    summed = acc_ref[0] + acc_ref[1]
    cnt = cnt_ref[0][:, 0:1] + cnt_ref[1][:, 0:1]
    mean = summed / jnp.clip(cnt, 1.0, None)
    out = (jnp.dot(mean, wlT_ref[...], preferred_element_type=jnp.float32)
           + jnp.dot(x_ref[...], wrT_ref[...], preferred_element_type=jnp.float32)
           + b_ref[...])
    mu = jnp.mean(out, axis=-1, keepdims=True)
    var = jnp.mean((out - mu) ** 2, axis=-1, keepdims=True)
    out = (out - mu) / jnp.sqrt(var + 1e-5) * g_ref[...] + be_ref[...]
    o_ref[...] = jnp.maximum(out, 0.0)


def _tc_finish(acc, cnt, x, W_l, b_l, W_r, gamma, beta):
    n, d = x.shape
    blk = 2000
    grid = n // blk
    return pl.pallas_call(
        _tc_finish_body,
        grid=(grid,),
        in_specs=[
            pl.BlockSpec((NUM_CORES, blk, d), lambda i: (0, i, 0)),
            pl.BlockSpec((NUM_CORES, blk, CW), lambda i: (0, i, 0)),
            pl.BlockSpec((blk, d), lambda i: (i, 0)),
            pl.BlockSpec((d, d), lambda i: (0, 0)),
            pl.BlockSpec((d, d), lambda i: (0, 0)),
            pl.BlockSpec((1, d), lambda i: (0, 0)),
            pl.BlockSpec((1, d), lambda i: (0, 0)),
            pl.BlockSpec((1, d), lambda i: (0, 0)),
        ],
        out_specs=pl.BlockSpec((blk, d), lambda i: (i, 0)),
        out_shape=jax.ShapeDtypeStruct((n, d), jnp.float32),
    )(acc, cnt, x, W_l.T, W_r.T, b_l.reshape(1, d),
      gamma.reshape(1, d), beta.reshape(1, d))


def kernel(x, edge_index, W_l, b_l, W_r, gamma, beta):
    n, d = x.shape
    e = edge_index.shape[1]

    n_pad = -(-(n + 1) // K) * K

    epw = -(-e // (NW * G * K)) * (G * K)
    groups = epw // (G * K)
    e_pad = epw * NW
    pad = e_pad - e
    src = edge_index[0].astype(jnp.int32)
    dst = edge_index[1].astype(jnp.int32)
    if pad:
        spread = n + jnp.arange(pad, dtype=jnp.int32) % (n_pad - n)
        src = jnp.concatenate([src, jnp.arange(pad, dtype=jnp.int32) % n])
        dst = jnp.concatenate([dst, spread])
    src4 = src.reshape(NW, groups, G, K)
    dst4 = dst.reshape(NW, groups, G, K)

    acc = _sc_sum(x, src4, dst4, n_pad, groups)
    cnt = _sc_count(dst4, n_pad, groups)
    return _tc_finish(acc, cnt, x, W_l, b_l, W_r, gamma, beta)


# speedup vs baseline: 9.6529x; 1.0197x over previous
import functools

import jax
import jax.numpy as jnp
from jax import lax
from jax.experimental import pallas as pl
from jax.experimental.pallas import tpu as pltpu
from jax.experimental.pallas import tpu_sc as plsc

NUM_CORES = 2
NUM_SUBCORES = 16
NW = NUM_CORES * NUM_SUBCORES
K = 128
G = 16


def _row_chunks(rows_per_tile):
    chunks = []
    off = 0
    while off < rows_per_tile:
        sz = min(K, rows_per_tile - off)
        chunks.append((off, sz))
        off += sz
    return chunks


def _sc_aggregate(x, src4, dst4, n_pad, groups):
    n, d = x.shape
    rows_per_tile = n_pad // NUM_SUBCORES
    row_chunks = _row_chunks(rows_per_tile)
    z_rows = jnp.zeros((K, d), jnp.float32)
    o_rows = jnp.ones((K, K), jnp.float32)
    mesh = plsc.VectorSubcoreMesh(core_axis_name="c", subcore_axis_name="s")

    @functools.partial(
        pl.kernel,
        out_type=(
            jax.ShapeDtypeStruct((NUM_CORES, n_pad, d), jnp.float32),
            jax.ShapeDtypeStruct((NUM_CORES, n_pad, K), jnp.float32),
        ),
        mesh=mesh,
        scratch_types=[
            pltpu.VMEM((G, K), jnp.int32),
            pltpu.VMEM((G, K), jnp.int32),
            pltpu.VMEM((K, d), jnp.float32),
            pltpu.VMEM((K, d), jnp.float32),
            pltpu.VMEM_SHARED((n_pad, d), jnp.float32),
            pltpu.SemaphoreType.DMA,
            pltpu.SemaphoreType.DMA,
        ],
    )
    def sc_kernel(x_hbm, src_hbm, dst_hbm, zr_hbm, on_hbm,
                  acc_out, cnt_out,
                  src_v, dst_v, rows0_v, rows1_v, acc_sh, sem0, sem1):
        c = lax.axis_index("c")
        s = lax.axis_index("s")
        wid = s * NUM_CORES + c
        row0 = s * rows_per_tile
        bufs = (rows0_v, rows1_v)
        sems = (sem0, sem1)

        pltpu.sync_copy(zr_hbm, rows0_v)
        for off, sz in row_chunks:
            pltpu.sync_copy(rows0_v.at[pl.ds(0, sz)],
                            acc_sh.at[pl.ds(row0 + off, sz)])
        plsc.subcore_barrier()

        @pl.loop(0, groups)
        def _(g):
            pltpu.sync_copy(src_hbm.at[wid, g], src_v)
            pltpu.sync_copy(dst_hbm.at[wid, g], dst_v)
            pltpu.async_copy(x_hbm.at[src_v.at[0]], bufs[0], sems[0])
            for jj in range(G):
                if jj + 1 < G:
                    pltpu.async_copy(x_hbm.at[src_v.at[jj + 1]],
                                     bufs[(jj + 1) % 2], sems[(jj + 1) % 2])
                pltpu.make_async_copy(
                    x_hbm.at[src_v.at[jj]], bufs[jj % 2], sems[jj % 2]).wait()
                pltpu.sync_copy(bufs[jj % 2],
                                acc_sh.at[dst_v.at[jj]], add=True)

        plsc.subcore_barrier()

        for off, sz in row_chunks:
            pltpu.sync_copy(acc_sh.at[pl.ds(row0 + off, sz)],
                            rows0_v.at[pl.ds(0, sz)])
            pltpu.sync_copy(rows0_v.at[pl.ds(0, sz)],
                            acc_out.at[c, pl.ds(row0 + off, sz)])
        pltpu.sync_copy(zr_hbm, rows0_v)
        for off, sz in row_chunks:
            pltpu.sync_copy(rows0_v.at[pl.ds(0, sz)],
                            acc_sh.at[pl.ds(row0 + off, sz)])
        pltpu.sync_copy(on_hbm, rows1_v)
        plsc.subcore_barrier()

        @pl.loop(0, groups)
        def _(g):
            pltpu.sync_copy(dst_hbm.at[wid, g], dst_v)
            for jj in range(G):
                pltpu.sync_copy(rows1_v, acc_sh.at[dst_v.at[jj]], add=True)

        plsc.subcore_barrier()

        for off, sz in row_chunks:
            pltpu.sync_copy(acc_sh.at[pl.ds(row0 + off, sz)],
                            rows0_v.at[pl.ds(0, sz)])
            pltpu.sync_copy(rows0_v.at[pl.ds(0, sz)],
                            cnt_out.at[c, pl.ds(row0 + off, sz)])

    return sc_kernel(x, src4, dst4, z_rows, o_rows)


def _tc_finish_body(acc_ref, cnt_ref, x_ref, wlT_ref, wrT_ref, b_ref,
                    g_ref, be_ref, o_ref):
    summed = acc_ref[0] + acc_ref[1]
    cnt = cnt_ref[0][:, 0:1] + cnt_ref[1][:, 0:1]
    mean = summed / jnp.clip(cnt, 1.0, None)
    out = (jnp.dot(mean, wlT_ref[...], preferred_element_type=jnp.float32)
           + jnp.dot(x_ref[...], wrT_ref[...], preferred_element_type=jnp.float32)
           + b_ref[...])
    mu = jnp.mean(out, axis=-1, keepdims=True)
    var = jnp.mean((out - mu) ** 2, axis=-1, keepdims=True)
    out = (out - mu) / jnp.sqrt(var + 1e-5) * g_ref[...] + be_ref[...]
    o_ref[...] = jnp.maximum(out, 0.0)


def _tc_finish(acc, cnt, x, W_l, b_l, W_r, gamma, beta):
    n, d = x.shape
    blk = 2000
    grid = n // blk
    return pl.pallas_call(
        _tc_finish_body,
        grid=(grid,),
        in_specs=[
            pl.BlockSpec((NUM_CORES, blk, d), lambda i: (0, i, 0)),
            pl.BlockSpec((NUM_CORES, blk, K), lambda i: (0, i, 0)),
            pl.BlockSpec((blk, d), lambda i: (i, 0)),
            pl.BlockSpec((d, d), lambda i: (0, 0)),
            pl.BlockSpec((d, d), lambda i: (0, 0)),
            pl.BlockSpec((1, d), lambda i: (0, 0)),
            pl.BlockSpec((1, d), lambda i: (0, 0)),
            pl.BlockSpec((1, d), lambda i: (0, 0)),
        ],
        out_specs=pl.BlockSpec((blk, d), lambda i: (i, 0)),
        out_shape=jax.ShapeDtypeStruct((n, d), jnp.float32),
    )(acc, cnt, x, W_l.T, W_r.T, b_l.reshape(1, d),
      gamma.reshape(1, d), beta.reshape(1, d))


def kernel(x, edge_index, W_l, b_l, W_r, gamma, beta):
    n, d = x.shape
    e = edge_index.shape[1]

    n_pad = -(-(n + 1) // K) * K

    epw = -(-e // (NW * G * K)) * (G * K)
    groups = epw // (G * K)
    e_pad = epw * NW
    pad = e_pad - e
    src = edge_index[0].astype(jnp.int32)
    dst = edge_index[1].astype(jnp.int32)
    if pad:
        spread = n + jnp.arange(pad, dtype=jnp.int32) % (n_pad - n)
        src = jnp.concatenate([src, jnp.arange(pad, dtype=jnp.int32) % n])
        dst = jnp.concatenate([dst, spread])
    src4 = src.reshape(NW, groups, G, K)
    dst4 = dst.reshape(NW, groups, G, K)

    acc, cnt = _sc_aggregate(x, src4, dst4, n_pad, groups)
    return _tc_finish(acc, cnt, x, W_l, b_l, W_r, gamma, beta)


# speedup vs baseline: 9.7042x; 1.0053x over previous
import functools

import jax
import jax.numpy as jnp
from jax import lax
from jax.experimental import pallas as pl
from jax.experimental.pallas import tpu as pltpu
from jax.experimental.pallas import tpu_sc as plsc

NUM_CORES = 2
NUM_SUBCORES = 16
NW = NUM_CORES * NUM_SUBCORES
K = 128
G = 16


def _row_chunks(rows_per_tile):
    chunks = []
    off = 0
    while off < rows_per_tile:
        sz = min(K, rows_per_tile - off)
        chunks.append((off, sz))
        off += sz
    return chunks


def _sc_aggregate(x, src4, dst4, n_pad, groups):
    n, d = x.shape
    rows_per_tile = n_pad // NUM_SUBCORES
    row_chunks = _row_chunks(rows_per_tile)
    z_rows = jnp.zeros((K, d), jnp.float32)
    o_rows = jnp.ones((K, K), jnp.float32)
    mesh = plsc.VectorSubcoreMesh(core_axis_name="c", subcore_axis_name="s")

    @functools.partial(
        pl.kernel,
        out_type=(
            jax.ShapeDtypeStruct((NUM_CORES, n_pad, d), jnp.float32),
            jax.ShapeDtypeStruct((NUM_CORES, n_pad, K), jnp.float32),
        ),
        mesh=mesh,
        scratch_types=[
            pltpu.VMEM((G, K), jnp.int32),
            pltpu.VMEM((G, K), jnp.int32),
            pltpu.VMEM((K, d), jnp.float32),
            pltpu.VMEM((K, d), jnp.float32),
            pltpu.VMEM_SHARED((n_pad, d), jnp.float32),
            pltpu.SemaphoreType.DMA,
            pltpu.SemaphoreType.DMA,
        ],
    )
    def sc_kernel(x_hbm, src_hbm, dst_hbm, zr_hbm, on_hbm,
                  acc_out, cnt_out,
                  src_v, dst_v, rows0_v, rows1_v, acc_sh, sem0, sem1):
        c = lax.axis_index("c")
        s = lax.axis_index("s")
        wid = s * NUM_CORES + c
        row0 = s * rows_per_tile
        bufs = (rows0_v, rows1_v)
        sems = (sem0, sem1)

        pltpu.sync_copy(zr_hbm, rows0_v)
        for off, sz in row_chunks:
            pltpu.sync_copy(rows0_v.at[pl.ds(0, sz)],
                            acc_sh.at[pl.ds(row0 + off, sz)])
        plsc.subcore_barrier()

        @pl.loop(0, groups)
        def _(g):
            pltpu.sync_copy(src_hbm.at[wid, g], src_v)
            pltpu.sync_copy(dst_hbm.at[wid, g], dst_v)
            pltpu.async_copy(x_hbm.at[src_v.at[0]], bufs[0], sems[0])
            for jj in range(G):
                if jj + 1 < G:
                    pltpu.async_copy(x_hbm.at[src_v.at[jj + 1]],
                                     bufs[(jj + 1) % 2], sems[(jj + 1) % 2])
                pltpu.make_async_copy(
                    x_hbm.at[src_v.at[jj]], bufs[jj % 2], sems[jj % 2]).wait()
                pltpu.sync_copy(bufs[jj % 2],
                                acc_sh.at[dst_v.at[jj]], add=True)

        plsc.subcore_barrier()

        for off, sz in row_chunks:
            pltpu.sync_copy(acc_sh.at[pl.ds(row0 + off, sz)],
                            rows0_v.at[pl.ds(0, sz)])
            pltpu.sync_copy(rows0_v.at[pl.ds(0, sz)],
                            acc_out.at[c, pl.ds(row0 + off, sz)])
        pltpu.sync_copy(zr_hbm, rows0_v)
        for off, sz in row_chunks:
            pltpu.sync_copy(rows0_v.at[pl.ds(0, sz)],
                            acc_sh.at[pl.ds(row0 + off, sz)])
        pltpu.sync_copy(on_hbm, rows1_v)
        plsc.subcore_barrier()

        @pl.loop(0, groups)
        def _(g):
            pltpu.sync_copy(dst_hbm.at[wid, g], dst_v)
            for jj in range(G):
                pltpu.async_copy(rows1_v, acc_sh.at[dst_v.at[jj]],
                                 sem0, add=True)
            for jj in range(G):
                pltpu.make_async_copy(rows1_v, acc_sh.at[dst_v.at[jj]],
                                      sem0).wait()

        plsc.subcore_barrier()

        for off, sz in row_chunks:
            pltpu.sync_copy(acc_sh.at[pl.ds(row0 + off, sz)],
                            rows0_v.at[pl.ds(0, sz)])
            pltpu.sync_copy(rows0_v.at[pl.ds(0, sz)],
                            cnt_out.at[c, pl.ds(row0 + off, sz)])

    return sc_kernel(x, src4, dst4, z_rows, o_rows)


def _tc_finish_body(acc_ref, cnt_ref, x_ref, wlT_ref, wrT_ref, b_ref,
                    g_ref, be_ref, o_ref):
    summed = acc_ref[0] + acc_ref[1]
    cnt = cnt_ref[0][:, 0:1] + cnt_ref[1][:, 0:1]
    mean = summed / jnp.clip(cnt, 1.0, None)
    out = (jnp.dot(mean, wlT_ref[...], preferred_element_type=jnp.float32)
           + jnp.dot(x_ref[...], wrT_ref[...], preferred_element_type=jnp.float32)
           + b_ref[...])
    mu = jnp.mean(out, axis=-1, keepdims=True)
    var = jnp.mean((out - mu) ** 2, axis=-1, keepdims=True)
    out = (out - mu) / jnp.sqrt(var + 1e-5) * g_ref[...] + be_ref[...]
    o_ref[...] = jnp.maximum(out, 0.0)


def _tc_finish(acc, cnt, x, W_l, b_l, W_r, gamma, beta):
    n, d = x.shape
    blk = 2000
    grid = n // blk
    return pl.pallas_call(
        _tc_finish_body,
        grid=(grid,),
        in_specs=[
            pl.BlockSpec((NUM_CORES, blk, d), lambda i: (0, i, 0)),
            pl.BlockSpec((NUM_CORES, blk, K), lambda i: (0, i, 0)),
            pl.BlockSpec((blk, d), lambda i: (i, 0)),
            pl.BlockSpec((d, d), lambda i: (0, 0)),
            pl.BlockSpec((d, d), lambda i: (0, 0)),
            pl.BlockSpec((1, d), lambda i: (0, 0)),
            pl.BlockSpec((1, d), lambda i: (0, 0)),
            pl.BlockSpec((1, d), lambda i: (0, 0)),
        ],
        out_specs=pl.BlockSpec((blk, d), lambda i: (i, 0)),
        out_shape=jax.ShapeDtypeStruct((n, d), jnp.float32),
    )(acc, cnt, x, W_l.T, W_r.T, b_l.reshape(1, d),
      gamma.reshape(1, d), beta.reshape(1, d))


def kernel(x, edge_index, W_l, b_l, W_r, gamma, beta):
    n, d = x.shape
    e = edge_index.shape[1]

    n_pad = -(-(n + 1) // K) * K

    epw = -(-e // (NW * G * K)) * (G * K)
    groups = epw // (G * K)
    e_pad = epw * NW
    pad = e_pad - e
    src = edge_index[0].astype(jnp.int32)
    dst = edge_index[1].astype(jnp.int32)
    if pad:
        spread = n + jnp.arange(pad, dtype=jnp.int32) % (n_pad - n)
        src = jnp.concatenate([src, jnp.arange(pad, dtype=jnp.int32) % n])
        dst = jnp.concatenate([dst, spread])
    src4 = src.reshape(NW, groups, G, K)
    dst4 = dst.reshape(NW, groups, G, K)

    acc, cnt = _sc_aggregate(x, src4, dst4, n_pad, groups)
    return _tc_finish(acc, cnt, x, W_l, b_l, W_r, gamma, beta)
